# baseline (speedup 1.0000x reference)
import functools

import jax
import jax.numpy as jnp
from jax import lax
from jax.experimental import pallas as pl
from jax.experimental.pallas import tpu as pltpu
from jax.experimental.pallas import tpu_sc as plsc

D = 64
DP = 128
L = 20
NW = 32
NC = 2
GB = 16


def _rsqrt(x):
    i = plsc.bitcast(x, jnp.int32)
    i = jnp.int32(0x5F3759DF) - lax.shift_right_logical(i, jnp.int32(1))
    y = plsc.bitcast(i, jnp.float32)
    for _ in range(3):
        y = y * (jnp.float32(1.5) - jnp.float32(0.5) * x * y * y)
    return y


@functools.lru_cache(maxsize=None)
def _make_sc_kernel(B):
    NB = B // NW
    NG = NB // GB
    CROWS = GB * L
    mesh = plsc.VectorSubcoreMesh(core_axis_name="c", subcore_axis_name="s")

    @functools.partial(
        pl.kernel,
        out_type=jax.ShapeDtypeStruct((B * L,), jnp.float32),
        mesh=mesh,
        compiler_params=pltpu.CompilerParams(
            needs_layout_passes=False, use_tc_tiling_on_sc=True),
        scratch_types=[
            pltpu.VMEM((NB,), jnp.int32),
            pltpu.VMEM((L * NB,), jnp.int32),
            pltpu.VMEM((L * NB,), jnp.int32),
            pltpu.VMEM((2, GB, DP), jnp.float32),
            pltpu.VMEM((2, CROWS, DP), jnp.float32),
            pltpu.VMEM((NB * L,), jnp.float32),
            pltpu.SemaphoreType.DMA((2,)),
        ],
    )
    def body(t_hbm, c_hbm, ti_hbm, ci_hbm, out_hbm, ti_v, ci_v, ci2_v, tr_v,
             cr_v, out_v, sem):
        wid = lax.axis_index("s") * NC + lax.axis_index("c")
        pltpu.sync_copy(ti_hbm.at[pl.ds(wid * NB, NB)], ti_v)
        for l in range(L):
            pltpu.sync_copy(ci_hbm.at[pl.ds(l * B + wid * NB, NB)],
                            ci_v.at[pl.ds(l * NB, NB)])

        def halve(i, carry):
            v = ci_v[pl.ds(i * 16, 16)]
            ci2_v[pl.ds(i * 16, 16)] = lax.shift_right_logical(v, 1)
            return carry

        lax.fori_loop(0, L * NB // 16, halve, 0)
        lanes = lax.iota(jnp.int32, 16)
        one = jnp.int32(1)

        def start(g, p):
            s = sem.at[p]
            tvi = lax.shift_right_logical(ti_v[pl.ds(g * GB, 16)], 1)
            pltpu.async_copy(t_hbm.at[tvi], tr_v.at[p], s)
            cdst = cr_v.at[p]
            for l in range(L):
                pltpu.async_copy(
                    c_hbm.at[ci2_v.at[pl.ds(l * NB + g * GB, 16)]],
                    cdst.at[pl.ds(l * GB, GB)], s)

        def wait(p):
            pltpu.make_async_copy(t_hbm.at[pl.ds(0, GB), :],
                                  tr_v.at[p], sem.at[p]).wait()
            pltpu.make_async_copy(c_hbm.at[pl.ds(0, CROWS), :],
                                  cr_v.at[p], sem.at[p]).wait()

        start(0, 0)

        def group(g, carry):
            p = lax.rem(g, 2)
            wait(p)

            @pl.when(g + 1 < NG)
            def _():
                start(g + 1, 1 - p)

            tr = tr_v.at[p]
            cr = cr_v.at[p]
            LC = L // 2
            row16 = lanes
            out_base = g * CROWS + row16 * L
            toff = lax.shift_left(ti_v[pl.ds(g * GB, 16)] & one, 6)
            scale_t = jnp.float32(1.0)
            for lc in range(L // LC):
                coffs = [
                    lax.shift_left(
                        ci_v[pl.ds((lc * LC + jj) * NB + g * GB, 16)] & one, 6)
                    for jj in range(LC)]

                def dblk(dc, acc, lc=lc, coffs=coffs):
                    ss_t, accd, accs = acc
                    accd, accs = list(accd), list(accs)
                    col0 = dc * 16
                    for dd in range(16):
                        col = jnp.full((16,), col0 + dd, jnp.int32)
                        tv = plsc.load_gather(tr, [row16, toff + col])
                        if lc == 0:
                            ss_t = ss_t + tv * tv
                        for jj in range(LC):
                            cv = plsc.load_gather(
                                cr, [row16 + (lc * LC + jj) * GB,
                                     coffs[jj] + col])
                            accd[jj] = accd[jj] + tv * cv
                            accs[jj] = accs[jj] + cv * cv
                    return ss_t, tuple(accd), tuple(accs)

                z = jnp.zeros((16,), jnp.float32)
                zd = tuple(jnp.zeros((16,), jnp.float32) for _ in range(LC))
                zs = tuple(jnp.zeros((16,), jnp.float32) for _ in range(LC))
                ss_t, accd, accs = lax.fori_loop(
                    0, D // 16, dblk, (z, zd, zs))
                if lc == 0:
                    scale_t = jnp.minimum(jnp.float32(1.0), _rsqrt(ss_t))
                for jj in range(LC):
                    scale_c = jnp.minimum(jnp.float32(1.0), _rsqrt(accs[jj]))
                    val = accd[jj] * scale_t * scale_c
                    plsc.store_scatter(
                        out_v, [out_base + (lc * LC + jj)], val)
            return carry

        lax.fori_loop(0, NG, group, 0)
        pltpu.sync_copy(out_v, out_hbm.at[pl.ds(wid * NB * L, NB * L)])

    return body


def kernel(inputs, context, t_table, c_table):
    B = inputs.shape[0]
    V2 = (t_table.shape[0] - 1) // 2
    t2 = t_table[: 2 * V2].reshape(V2, 2 * D)
    c2 = c_table[: 2 * V2].reshape(V2, 2 * D)
    ti = inputs.astype(jnp.int32)
    ci = context.T.astype(jnp.int32).reshape(-1)
    out = _make_sc_kernel(B)(t2, c2, ti, ci)
    return out.reshape(B, L)


# speedup vs baseline: 1.3092x; 1.3092x over previous
import functools

import jax
import jax.numpy as jnp
from jax import lax
from jax.experimental import pallas as pl
from jax.experimental.pallas import tpu as pltpu
from jax.experimental.pallas import tpu_sc as plsc

D = 64
DP = 64
L = 20
NW = 32
NC = 2
GB = 16


def _rsqrt(x):
    i = plsc.bitcast(x, jnp.int32)
    i = jnp.int32(0x5F3759DF) - lax.shift_right_logical(i, jnp.int32(1))
    y = plsc.bitcast(i, jnp.float32)
    for _ in range(3):
        y = y * (jnp.float32(1.5) - jnp.float32(0.5) * x * y * y)
    return y


@functools.lru_cache(maxsize=None)
def _make_sc_kernel(B):
    NB = B // NW
    NG = NB // GB
    CROWS = GB * L
    mesh = plsc.VectorSubcoreMesh(core_axis_name="c", subcore_axis_name="s")

    @functools.partial(
        pl.kernel,
        out_type=jax.ShapeDtypeStruct((B * L,), jnp.float32),
        mesh=mesh,
        compiler_params=pltpu.CompilerParams(
            needs_layout_passes=False, use_tc_tiling_on_sc=True),
        scratch_types=[
            pltpu.VMEM((NB,), jnp.int32),
            pltpu.VMEM((L * NB,), jnp.int32),
            pltpu.VMEM((2, GB, DP), jnp.float32),
            pltpu.VMEM((2, CROWS, DP), jnp.float32),
            pltpu.VMEM((NB * L,), jnp.float32),
            pltpu.SemaphoreType.DMA((2,)),
        ],
    )
    def body(t_hbm, c_hbm, ti_hbm, ci_hbm, out_hbm, ti_v, ci_v, tr_v, cr_v,
             out_v, sem):
        wid = lax.axis_index("s") * NC + lax.axis_index("c")
        pltpu.sync_copy(ti_hbm.at[pl.ds(wid * NB, NB)], ti_v)
        for l in range(L):
            pltpu.sync_copy(ci_hbm.at[pl.ds(l * B + wid * NB, NB)],
                            ci_v.at[pl.ds(l * NB, NB)])
        lanes = lax.iota(jnp.int32, 16)

        def start(g, p):
            s = sem.at[p]
            tdst = tr_v.at[p]
            tvi = ti_v[pl.ds(g * GB, 16)]
            for q in range(16):
                pltpu.async_copy(t_hbm.at[pl.ds(tvi[q], 1), :],
                                 tdst.at[pl.ds(q, 1), :], s)
            cdst = cr_v.at[p]
            for l in range(L):
                cvi = ci_v[pl.ds(l * NB + g * GB, 16)]
                for q in range(16):
                    pltpu.async_copy(
                        c_hbm.at[pl.ds(cvi[q], 1), :],
                        cdst.at[pl.ds(l * GB + q, 1), :], s)

        def wait(p):
            pltpu.make_async_copy(t_hbm.at[pl.ds(0, GB), :],
                                  tr_v.at[p], sem.at[p]).wait()
            pltpu.make_async_copy(c_hbm.at[pl.ds(0, CROWS), :],
                                  cr_v.at[p], sem.at[p]).wait()

        start(0, 0)

        def group(g, carry):
            p = lax.rem(g, 2)

            @pl.when(g + 1 < NG)
            def _():
                start(g + 1, 1 - p)

            wait(p)

            tr = tr_v.at[p]
            cr = cr_v.at[p]
            LC = L // 2
            row16 = lanes
            out_base = g * CROWS + row16 * L
            scale_t = jnp.float32(1.0)
            for lc in range(L // LC):

                def dblk(dc, acc, lc=lc):
                    ss_t, accd, accs = acc
                    accd, accs = list(accd), list(accs)
                    col0 = dc * 16
                    for dd in range(16):
                        col = jnp.full((16,), col0 + dd, jnp.int32)
                        tv = plsc.load_gather(tr, [row16, col])
                        if lc == 0:
                            ss_t = ss_t + tv * tv
                        for jj in range(LC):
                            cv = plsc.load_gather(
                                cr, [row16 + (lc * LC + jj) * GB, col])
                            accd[jj] = accd[jj] + tv * cv
                            accs[jj] = accs[jj] + cv * cv
                    return ss_t, tuple(accd), tuple(accs)

                z = jnp.zeros((16,), jnp.float32)
                zd = tuple(jnp.zeros((16,), jnp.float32) for _ in range(LC))
                zs = tuple(jnp.zeros((16,), jnp.float32) for _ in range(LC))
                ss_t, accd, accs = lax.fori_loop(
                    0, D // 16, dblk, (z, zd, zs))
                if lc == 0:
                    scale_t = jnp.minimum(jnp.float32(1.0), _rsqrt(ss_t))
                for jj in range(LC):
                    scale_c = jnp.minimum(jnp.float32(1.0), _rsqrt(accs[jj]))
                    val = accd[jj] * scale_t * scale_c
                    plsc.store_scatter(
                        out_v, [out_base + (lc * LC + jj)], val)
            return carry

        lax.fori_loop(0, NG, group, 0)
        pltpu.sync_copy(out_v, out_hbm.at[pl.ds(wid * NB * L, NB * L)])

    return body


def kernel(inputs, context, t_table, c_table):
    B = inputs.shape[0]
    ti = inputs.astype(jnp.int32)
    ci = context.T.astype(jnp.int32).reshape(-1)
    out = _make_sc_kernel(B)(t_table, c_table, ti, ci)
    return out.reshape(B, L)
